# baseline (speedup 1.0000x reference)
import jax
import jax.numpy as jnp
from jax import lax
from jax.experimental import pallas as pl
from jax.experimental.pallas import tpu as pltpu
from jax.experimental.pallas import tpu_sc as plsc

BATCH = 16384
EMBED_DIM = 32
NROWS = 1000001
LANES = 16
CHUNK = 128
RATING_SCALE = 5.5

NT = 7813
NTFULL = NT - 1
NA = EMBED_DIM // 8
NBLK = NA * NT

_info = plsc.get_sparse_core_info()
_NC, _NS = _info.num_cores, _info.num_subcores
NW = _NC * _NS
BPW = BATCH // NW
NCHUNK = BPW // CHUNK
NGROUP = BPW // LANES
NVEC = CHUNK // LANES

_PER_W = NTFULL // NW
_EXTRA = NTFULL - _PER_W * NW


GRP = 8


def _repack(wut_hbm, wit_hbm, tail_u_hbm, tail_i_hbm,
            pu_hbm, pi_hbm,
            bigbuf, tbuf, fsem, wsem):
    wid = lax.axis_index("s") * _NC + lax.axis_index("c")
    lo = wid * _PER_W + jnp.minimum(wid, _EXTRA)
    cnt = _PER_W + jnp.where(wid < _EXTRA, 1, 0)
    hi = lo + cnt
    ngrp = cnt // GRP

    for src, dst in ((wut_hbm, pu_hbm), (wit_hbm, pi_hbm)):

        def fetch8(tbase, half, src=src):
            for k in range(GRP):
                off = pl.multiple_of((tbase + k) * CHUNK, CHUNK)
                pltpu.async_copy(src.at[:, pl.ds(off, CHUNK)],
                                 bigbuf.at[half * GRP + k], fsem)

        def drain_f(src=src):
            pltpu.make_async_copy(src.at[:, pl.ds(0, CHUNK)],
                                  bigbuf.at[0], fsem).wait()

        def write_batched(half, tbase, dst=dst):
            for a in range(NA):
                pltpu.async_copy(
                    bigbuf.at[pl.ds(half * GRP, GRP), pl.ds(8 * a, 8), :],
                    dst.at[pl.ds(a * NT + tbase, GRP)], wsem)

        def drain_wgroup(dst=dst):
            for a in range(NA):
                pltpu.make_async_copy(
                    dst.at[pl.ds(0, GRP)],
                    bigbuf.at[pl.ds(0, GRP), pl.ds(0, 8), :], wsem).wait()

        @pl.when(ngrp > 0)
        def _prime():
            fetch8(lo, 0)

        def step(gq, carry):
            for h in range(2):
                g = gq * 2 + h
                tbase = lo + g * GRP

                @pl.when(g < ngrp)
                def _do(h=h, g=g, tbase=tbase):
                    for _ in range(GRP):
                        drain_f()

                    @pl.when(g + 1 < ngrp)
                    def _pref(h=h, g=g, tbase=tbase):
                        @pl.when(g >= 1)
                        def _dw():
                            drain_wgroup()

                        fetch8(tbase + GRP, (h + 1) % 2)

                    write_batched(h, tbase)
            return carry

        lax.fori_loop(0, (ngrp + 1) // 2, step, 0, unroll=False)

        @pl.when(ngrp > 0)
        def _dw1():
            drain_wgroup()

        @pl.when(ngrp > 1)
        def _dw2():
            drain_wgroup()

        def tail_step(t, carry, src=src, dst=dst):
            off = pl.multiple_of(t * CHUNK, CHUNK)
            pltpu.async_copy(src.at[:, pl.ds(off, CHUNK)], bigbuf.at[0], fsem)
            drain_f()
            for a in range(NA):
                pltpu.async_copy(bigbuf.at[0].at[pl.ds(8 * a, 8), :],
                                 dst.at[a * NT + t], wsem)
            pltpu.make_async_copy(src.at[:, pl.ds(0, CHUNK)],
                                  bigbuf.at[0], wsem).wait()
            return carry

        lax.fori_loop(lo + ngrp * GRP, hi, tail_step, 0, unroll=False)

    @pl.when(wid == NW - 1)
    def _tail():
        for tsrc, dst in ((tail_u_hbm, pu_hbm), (tail_i_hbm, pi_hbm)):
            for a in range(NA):
                pltpu.sync_copy(tsrc.at[a], tbuf)
                pltpu.sync_copy(tbuf, dst.at[a * NT + NTFULL])


def _gather_compute(users_hbm, items_hbm, puf_hbm, pif_hbm, bu_hbm, bi_hbm,
                    out_hbm,
                    u_idx, i_idx, u_shift, i_shift, u_base, i_base,
                    u_cols, i_cols, u_b, i_b, out_v, sem, bsem):
    wid = lax.axis_index("s") * _NC + lax.axis_index("c")
    base = wid * BPW

    pltpu.sync_copy(users_hbm.at[pl.ds(wid * NCHUNK, NCHUNK)], u_idx)
    pltpu.sync_copy(items_hbm.at[pl.ds(wid * NCHUNK, NCHUNK)], i_idx)

    bias_handles = []
    for j in range(NCHUNK):
        sl = pl.ds(j * CHUNK, CHUNK)
        bias_handles.append(pltpu.async_copy(bu_hbm.at[u_idx.at[j]], u_b.at[sl], bsem))
        bias_handles.append(pltpu.async_copy(bi_hbm.at[i_idx.at[j]], i_b.at[sl], bsem))

    def mkbase(j, carry):
        for v in range(NVEC):
            sl = pl.ds(v * LANES, LANES)
            uv = u_idx[j, sl]
            iv = i_idx[j, sl]
            u_base[j, sl] = (uv >> 7) * 1024 + (uv & 127)
            i_base[j, sl] = (iv >> 7) * 1024 + (iv & 127)
        return carry

    lax.fori_loop(0, NCHUNK, mkbase, 0, unroll=False)

    def fetch(d, carry):
        cd = (d // 8) * NT * 1024 + (d % 8) * CHUNK
        for j in range(NCHUNK):
            for v in range(NVEC):
                sl = pl.ds(v * LANES, LANES)
                u_shift[d, j, sl] = u_base[j, sl] + cd
                i_shift[d, j, sl] = i_base[j, sl] + cd
        for j in range(NCHUNK):
            sl = pl.ds(d * BPW + j * CHUNK, CHUNK)
            pltpu.async_copy(puf_hbm.at[u_shift.at[d].at[j]], u_cols.at[sl], sem)
            pltpu.async_copy(pif_hbm.at[i_shift.at[d].at[j]], i_cols.at[sl], sem)
        return carry

    lax.fori_loop(0, EMBED_DIM, fetch, 0, unroll=False)

    pltpu.make_async_copy(puf_hbm.at[pl.ds(0, EMBED_DIM * BPW)], u_cols, sem).wait()
    pltpu.make_async_copy(pif_hbm.at[pl.ds(0, EMBED_DIM * BPW)], i_cols, sem).wait()

    for h in bias_handles:
        h.wait()

    def group(g, carry):
        sl = pl.ds(g * LANES, LANES)
        acc = u_b[sl] + i_b[sl]
        for d in range(EMBED_DIM):
            dsl = pl.ds(d * BPW + g * LANES, LANES)
            acc = acc + u_cols[dsl] * i_cols[dsl]
        out_v[sl] = RATING_SCALE / (1.0 + jnp.exp(-acc))
        return carry

    lax.fori_loop(0, NGROUP, group, 0, unroll=False)

    pltpu.sync_copy(out_v, out_hbm.at[pl.ds(base, BPW)])


def kernel(users, items, W_user, W_item, B_user, B_item):
    u = users.reshape(BATCH // CHUNK, CHUNK).astype(jnp.int32)
    it = items.reshape(BATCH // CHUNK, CHUNK).astype(jnp.int32)
    mesh = plsc.VectorSubcoreMesh(core_axis_name="c", subcore_axis_name="s")

    ntail = NROWS - NTFULL * CHUNK
    def tailblock(W):
        t = lax.slice(W, (NTFULL * CHUNK, 0), (NROWS, EMBED_DIM))
        t = jnp.pad(t, ((0, CHUNK - ntail), (0, 0))).T
        return t.reshape(NA, 8, CHUNK)

    repack = pl.kernel(
        _repack,
        out_type=(jax.ShapeDtypeStruct((NBLK, 8, CHUNK), jnp.float32),
                  jax.ShapeDtypeStruct((NBLK, 8, CHUNK), jnp.float32)),
        mesh=mesh,
        compiler_params=pltpu.CompilerParams(
            needs_layout_passes=False, use_tc_tiling_on_sc=True),
        scratch_types=[
            pltpu.VMEM((2 * GRP, EMBED_DIM, CHUNK), jnp.float32),
            pltpu.VMEM((8, CHUNK), jnp.float32),
            pltpu.SemaphoreType.DMA,
            pltpu.SemaphoreType.DMA,
        ],
    )
    pu, pi = repack(W_user.T, W_item.T,
                    tailblock(W_user), tailblock(W_item))

    gather = pl.kernel(
        _gather_compute,
        out_type=jax.ShapeDtypeStruct((BATCH,), jnp.float32),
        mesh=mesh,
        compiler_params=pltpu.CompilerParams(
            needs_layout_passes=False, use_tc_tiling_on_sc=False),
        scratch_types=[
            pltpu.VMEM((NCHUNK, CHUNK), jnp.int32),
            pltpu.VMEM((NCHUNK, CHUNK), jnp.int32),
            pltpu.VMEM((EMBED_DIM, NCHUNK, CHUNK), jnp.int32),
            pltpu.VMEM((EMBED_DIM, NCHUNK, CHUNK), jnp.int32),
            pltpu.VMEM((NCHUNK, CHUNK), jnp.int32),
            pltpu.VMEM((NCHUNK, CHUNK), jnp.int32),
            pltpu.VMEM((EMBED_DIM * BPW,), jnp.float32),
            pltpu.VMEM((EMBED_DIM * BPW,), jnp.float32),
            pltpu.VMEM((BPW,), jnp.float32),
            pltpu.VMEM((BPW,), jnp.float32),
            pltpu.VMEM((BPW,), jnp.float32),
            pltpu.SemaphoreType.DMA,
            pltpu.SemaphoreType.DMA,
        ],
    )
    return gather(u, it, pu.reshape(-1), pi.reshape(-1),
                  B_user.reshape(-1), B_item.reshape(-1))


# speedup vs baseline: 1.0254x; 1.0254x over previous
import jax
import jax.numpy as jnp
from jax import lax
from jax.experimental import pallas as pl
from jax.experimental.pallas import tpu as pltpu
from jax.experimental.pallas import tpu_sc as plsc

BATCH = 16384
EMBED_DIM = 32
NROWS = 1000001
LANES = 16
CHUNK = 128
RATING_SCALE = 5.5

NT = 7813
NTFULL = NT - 1
NA = EMBED_DIM // 8
NBLK = NA * NT

_info = plsc.get_sparse_core_info()
_NC, _NS = _info.num_cores, _info.num_subcores
NW = _NC * _NS
BPW = BATCH // NW
NCHUNK = BPW // CHUNK
NGROUP = BPW // LANES
NVEC = CHUNK // LANES

_PER_W = NTFULL // NW
_EXTRA = NTFULL - _PER_W * NW


NBUF = 16


def _repack(wut_hbm, wit_hbm, tail_u_hbm, tail_i_hbm,
            pu_hbm, pi_hbm,
            *refs):
    bufs = refs[:NBUF]
    tbuf, fsem, wsem = refs[NBUF:]
    buf0 = bufs[0]
    wid = lax.axis_index("s") * _NC + lax.axis_index("c")
    lo = wid * _PER_W + jnp.minimum(wid, _EXTRA)
    cnt = _PER_W + jnp.where(wid < _EXTRA, 1, 0)
    hi = lo + cnt

    for src, dst in ((wut_hbm, pu_hbm), (wit_hbm, pi_hbm)):

        def fetch(t, buf, src=src):
            off = pl.multiple_of(t * CHUNK, CHUNK)
            pltpu.async_copy(src.at[:, pl.ds(off, CHUNK)], buf, fsem)

        def drain_f(src=src):
            pltpu.make_async_copy(src.at[:, pl.ds(0, CHUNK)], buf0, fsem).wait()

        def drain_w(src=src):
            pltpu.make_async_copy(src.at[:, pl.ds(0, CHUNK)], buf0, wsem).wait()

        def write4(buf, t, dst=dst):
            for a in range(NA):
                pltpu.async_copy(buf.at[pl.ds(8 * a, 8), :], dst.at[a * NT + t], wsem)

        for k in range(NBUF - 1):
            @pl.when(cnt > k)
            def _prime(k=k):
                fetch(lo + k, bufs[k])

        def step(q, carry):
            for k in range(NBUF):
                tk = lo + q * NBUF + k

                @pl.when(tk < hi)
                def _do(k=k, tk=tk):
                    drain_f()
                    write4(bufs[k], tk)
                    nxt = tk + (NBUF - 1)

                    @pl.when(nxt < hi)
                    def _refetch(k=k, nxt=nxt):
                        @pl.when(nxt - NBUF >= lo)
                        def _dw():
                            drain_w()

                        fetch(nxt, bufs[(k + NBUF - 1) % NBUF])
            return carry

        lax.fori_loop(0, (cnt + NBUF - 1) // NBUF, step, 0, unroll=False)

        for k in range(NBUF):
            @pl.when(cnt > k)
            def _dw_final(k=k):
                drain_w()

    @pl.when(wid == NW - 1)
    def _tail():
        for tsrc, dst in ((tail_u_hbm, pu_hbm), (tail_i_hbm, pi_hbm)):
            for a in range(NA):
                pltpu.sync_copy(tsrc.at[a], tbuf)
                pltpu.sync_copy(tbuf, dst.at[a * NT + NTFULL])


def _gather_compute(users_hbm, items_hbm, puf_hbm, pif_hbm, bu_hbm, bi_hbm,
                    out_hbm,
                    u_idx, i_idx, u_shift, i_shift, u_base, i_base,
                    u_cols, i_cols, u_b, i_b, out_v, sem, bsem):
    wid = lax.axis_index("s") * _NC + lax.axis_index("c")
    base = wid * BPW

    pltpu.sync_copy(users_hbm.at[pl.ds(wid * NCHUNK, NCHUNK)], u_idx)
    pltpu.sync_copy(items_hbm.at[pl.ds(wid * NCHUNK, NCHUNK)], i_idx)

    bias_handles = []
    for j in range(NCHUNK):
        sl = pl.ds(j * CHUNK, CHUNK)
        bias_handles.append(pltpu.async_copy(bu_hbm.at[u_idx.at[j]], u_b.at[sl], bsem))
        bias_handles.append(pltpu.async_copy(bi_hbm.at[i_idx.at[j]], i_b.at[sl], bsem))

    def mkbase(j, carry):
        for v in range(NVEC):
            sl = pl.ds(v * LANES, LANES)
            uv = u_idx[j, sl]
            iv = i_idx[j, sl]
            u_base[j, sl] = (uv >> 7) * 1024 + (uv & 127)
            i_base[j, sl] = (iv >> 7) * 1024 + (iv & 127)
        return carry

    lax.fori_loop(0, NCHUNK, mkbase, 0, unroll=False)

    def fetch(d, carry):
        cd = (d // 8) * NT * 1024 + (d % 8) * CHUNK
        for j in range(NCHUNK):
            for v in range(NVEC):
                sl = pl.ds(v * LANES, LANES)
                u_shift[d, j, sl] = u_base[j, sl] + cd
                i_shift[d, j, sl] = i_base[j, sl] + cd
        for j in range(NCHUNK):
            sl = pl.ds(d * BPW + j * CHUNK, CHUNK)
            pltpu.async_copy(puf_hbm.at[u_shift.at[d].at[j]], u_cols.at[sl], sem)
            pltpu.async_copy(pif_hbm.at[i_shift.at[d].at[j]], i_cols.at[sl], sem)
        return carry

    lax.fori_loop(0, EMBED_DIM, fetch, 0, unroll=False)

    pltpu.make_async_copy(puf_hbm.at[pl.ds(0, EMBED_DIM * BPW)], u_cols, sem).wait()
    pltpu.make_async_copy(pif_hbm.at[pl.ds(0, EMBED_DIM * BPW)], i_cols, sem).wait()

    for h in bias_handles:
        h.wait()

    def group(g, carry):
        sl = pl.ds(g * LANES, LANES)
        acc = u_b[sl] + i_b[sl]
        for d in range(EMBED_DIM):
            dsl = pl.ds(d * BPW + g * LANES, LANES)
            acc = acc + u_cols[dsl] * i_cols[dsl]
        out_v[sl] = RATING_SCALE / (1.0 + jnp.exp(-acc))
        return carry

    lax.fori_loop(0, NGROUP, group, 0, unroll=False)

    pltpu.sync_copy(out_v, out_hbm.at[pl.ds(base, BPW)])


def kernel(users, items, W_user, W_item, B_user, B_item):
    u = users.reshape(BATCH // CHUNK, CHUNK).astype(jnp.int32)
    it = items.reshape(BATCH // CHUNK, CHUNK).astype(jnp.int32)
    mesh = plsc.VectorSubcoreMesh(core_axis_name="c", subcore_axis_name="s")

    ntail = NROWS - NTFULL * CHUNK
    def tailblock(W):
        t = lax.slice(W, (NTFULL * CHUNK, 0), (NROWS, EMBED_DIM))
        t = jnp.pad(t, ((0, CHUNK - ntail), (0, 0))).T
        return t.reshape(NA, 8, CHUNK)

    repack = pl.kernel(
        _repack,
        out_type=(jax.ShapeDtypeStruct((NBLK, 8, CHUNK), jnp.float32),
                  jax.ShapeDtypeStruct((NBLK, 8, CHUNK), jnp.float32)),
        mesh=mesh,
        compiler_params=pltpu.CompilerParams(
            needs_layout_passes=False, use_tc_tiling_on_sc=True),
        scratch_types=(
            [pltpu.VMEM((EMBED_DIM, CHUNK), jnp.float32) for _ in range(NBUF)]
            + [
                pltpu.VMEM((8, CHUNK), jnp.float32),
                pltpu.SemaphoreType.DMA,
                pltpu.SemaphoreType.DMA,
            ]
        ),
    )
    pu, pi = repack(W_user.T, W_item.T,
                    tailblock(W_user), tailblock(W_item))

    gather = pl.kernel(
        _gather_compute,
        out_type=jax.ShapeDtypeStruct((BATCH,), jnp.float32),
        mesh=mesh,
        compiler_params=pltpu.CompilerParams(
            needs_layout_passes=False, use_tc_tiling_on_sc=False),
        scratch_types=[
            pltpu.VMEM((NCHUNK, CHUNK), jnp.int32),
            pltpu.VMEM((NCHUNK, CHUNK), jnp.int32),
            pltpu.VMEM((EMBED_DIM, NCHUNK, CHUNK), jnp.int32),
            pltpu.VMEM((EMBED_DIM, NCHUNK, CHUNK), jnp.int32),
            pltpu.VMEM((NCHUNK, CHUNK), jnp.int32),
            pltpu.VMEM((NCHUNK, CHUNK), jnp.int32),
            pltpu.VMEM((EMBED_DIM * BPW,), jnp.float32),
            pltpu.VMEM((EMBED_DIM * BPW,), jnp.float32),
            pltpu.VMEM((BPW,), jnp.float32),
            pltpu.VMEM((BPW,), jnp.float32),
            pltpu.VMEM((BPW,), jnp.float32),
            pltpu.SemaphoreType.DMA,
            pltpu.SemaphoreType.DMA,
        ],
    )
    return gather(u, it, pu.reshape(-1), pi.reshape(-1),
                  B_user.reshape(-1), B_item.reshape(-1))
